# Initial kernel scaffold; baseline (speedup 1.0000x reference)
import jax
import jax.numpy as jnp
from jax.experimental import pallas as pl


def kernel(h, edge_index, e, emb, Wl0, bl0, Wr0, g0, be0, Wl1, bl1, Wr1, g1, be1, Wl2, bl2, Wr2, g2, be2, Wl3, bl3, Wr3, g3, be3, Wm0, bm0, Wm1, bm1, Wm2, bm2):
    raise NotImplementedError("write your pallas kernel here")


# speedup vs baseline: 3.5426x; 3.5426x over previous
import functools

import jax
import jax.numpy as jnp
from jax import lax
from jax.experimental import pallas as pl
from jax.experimental.pallas import tpu as pltpu
from jax.experimental.pallas import tpu_sc as plsc

N = 10000
E = 320000
D = 128
NCORE = 2
NSUB = 16
NWORK = NCORE * NSUB
CH = 128
NCHUNK = 79
EPW = CH * NCHUNK
E_PAD = EPW * NWORK
N_PAD = 10240
ROWS_PT = N_PAD // NSUB

_MESH = plsc.VectorSubcoreMesh(core_axis_name="c", subcore_axis_name="s",
                               num_cores=NCORE, num_subcores=NSUB)


def _agg_body(x_hbm, src_hbm, dst_hbm, out_hbm, zb, src_v, dst_v, rows_v, acc_sh, sem):
    c = lax.axis_index("c")
    s = lax.axis_index("s")
    wid = c * NSUB + s
    z16 = jnp.zeros((16,), jnp.float32)
    for i in range(16):
        for j in range(8):
            zb[i, pl.ds(j * 16, 16)] = z16
    rbase = s * ROWS_PT

    def zero_body(k, carry):
        pltpu.sync_copy(zb, acc_sh.at[pl.ds(rbase + k * 16, 16)])
        return carry

    lax.fori_loop(0, ROWS_PT // 16, zero_body, 0)
    plsc.subcore_barrier()

    ebase0 = wid * EPW

    def edge_body(j, carry):
        eb = pl.multiple_of(ebase0 + j * CH, CH)
        pltpu.sync_copy(src_hbm.at[pl.ds(eb, CH)], src_v)
        pltpu.sync_copy(dst_hbm.at[pl.ds(eb, CH)], dst_v)
        pltpu.async_copy(x_hbm.at[src_v], rows_v, sem).wait()
        pltpu.sync_copy(rows_v, acc_sh.at[dst_v], add=True)
        return carry

    lax.fori_loop(0, NCHUNK, edge_body, 0)
    plsc.subcore_barrier()
    pltpu.sync_copy(acc_sh.at[pl.ds(rbase, ROWS_PT)],
                    out_hbm.at[c, pl.ds(rbase, ROWS_PT)])


_agg_call = pl.kernel(
    _agg_body,
    out_type=jax.ShapeDtypeStruct((NCORE, N_PAD, D), jnp.float32),
    mesh=_MESH,
    scratch_types=[
        pltpu.VMEM((16, D), jnp.float32),
        pltpu.VMEM((CH,), jnp.int32),
        pltpu.VMEM((CH,), jnp.int32),
        pltpu.VMEM((CH, D), jnp.float32),
        pltpu.VMEM_SHARED((N_PAD, D), jnp.float32),
        pltpu.SemaphoreType.DMA,
    ],
)


def _deg_body(dst_hbm, out_hbm, ones_v, zb, dst_v, dacc_sh):
    c = lax.axis_index("c")
    s = lax.axis_index("s")
    wid = c * NSUB + s
    one_row = jnp.where(lax.iota(jnp.int32, 16) == 0,
                        jnp.float32(1.0), jnp.float32(0.0))
    z16 = jnp.zeros((16,), jnp.float32)
    for i in range(CH):
        for j in range(8):
            ones_v[i, pl.ds(j * 16, 16)] = one_row if j == 0 else z16
    for i in range(16):
        for j in range(8):
            zb[i, pl.ds(j * 16, 16)] = z16
    rbase = s * ROWS_PT

    def zero_body(k, carry):
        pltpu.sync_copy(zb, dacc_sh.at[pl.ds(rbase + k * 16, 16)])
        return carry

    lax.fori_loop(0, ROWS_PT // 16, zero_body, 0)
    plsc.subcore_barrier()

    ebase0 = wid * EPW

    def edge_body(j, carry):
        eb = pl.multiple_of(ebase0 + j * CH, CH)
        pltpu.sync_copy(dst_hbm.at[pl.ds(eb, CH)], dst_v)
        pltpu.sync_copy(ones_v, dacc_sh.at[dst_v], add=True)
        return carry

    lax.fori_loop(0, NCHUNK, edge_body, 0)
    plsc.subcore_barrier()
    pltpu.sync_copy(dacc_sh.at[pl.ds(rbase, ROWS_PT)],
                    out_hbm.at[c, pl.ds(rbase, ROWS_PT)])


_deg_call = pl.kernel(
    _deg_body,
    out_type=jax.ShapeDtypeStruct((NCORE, N_PAD, D), jnp.float32),
    mesh=_MESH,
    scratch_types=[
        pltpu.VMEM((CH, D), jnp.float32),
        pltpu.VMEM((16, D), jnp.float32),
        pltpu.VMEM((CH,), jnp.int32),
        pltpu.VMEM_SHARED((N_PAD, D), jnp.float32),
    ],
)


def _embed_body(h_ref, emb_ref, out_ref):
    hh = h_ref[:]
    cols = lax.broadcasted_iota(jnp.int32, (1, D), 1)
    onehot = (hh == cols).astype(jnp.float32)
    out_ref[:] = jnp.dot(onehot, emb_ref[:], preferred_element_type=jnp.float32,
                         precision=lax.Precision.HIGHEST)


_embed_call = pl.pallas_call(
    _embed_body,
    out_shape=jax.ShapeDtypeStruct((N, D), jnp.float32),
)


def _dense_common(sacc_ref, dacc_ref, x_ref, wlt_ref, bl_ref, wrt_ref, g_ref, be_ref):
    ssum = (sacc_ref[0] + sacc_ref[1])[:N]
    deg = (dacc_ref[0] + dacc_ref[1])[:N, 0:1]
    rdeg = 1.0 / jnp.maximum(deg, 1.0)
    mean = ssum * rdeg
    x = x_ref[:]
    t = (jnp.dot(mean, wlt_ref[:], preferred_element_type=jnp.float32,
                 precision=lax.Precision.DEFAULT)
         + bl_ref[:]
         + jnp.dot(x, wrt_ref[:], preferred_element_type=jnp.float32,
                   precision=lax.Precision.DEFAULT))
    mu = jnp.mean(t, axis=0, keepdims=True)
    var = jnp.mean((t - mu) * (t - mu), axis=0, keepdims=True)
    return g_ref[:] * (t - mu) * lax.rsqrt(var + 1e-5) + be_ref[:] + x


def _dense_body(sacc_ref, dacc_ref, x_ref, wlt_ref, bl_ref, wrt_ref, g_ref, be_ref,
                out_ref):
    out_ref[:] = _dense_common(sacc_ref, dacc_ref, x_ref, wlt_ref, bl_ref,
                               wrt_ref, g_ref, be_ref)


_dense_call = pl.pallas_call(
    _dense_body,
    out_shape=jax.ShapeDtypeStruct((N, D), jnp.float32),
)


def _final_body(sacc_ref, dacc_ref, x_ref, wlt_ref, bl_ref, wrt_ref, g_ref, be_ref,
                wm0_ref, bm0_ref, wm1_ref, bm1_ref, wm2_ref, bm2_ref, out_ref):
    xo = _dense_common(sacc_ref, dacc_ref, x_ref, wlt_ref, bl_ref, wrt_ref,
                       g_ref, be_ref)
    y = jnp.maximum(jnp.dot(xo, wm0_ref[:], preferred_element_type=jnp.float32,
                            precision=lax.Precision.DEFAULT)
                    + bm0_ref[:], 0.0)
    y = jnp.maximum(jnp.dot(y, wm1_ref[:], preferred_element_type=jnp.float32,
                            precision=lax.Precision.DEFAULT)
                    + bm1_ref[:], 0.0)
    out_ref[:] = jnp.dot(y, wm2_ref[:], preferred_element_type=jnp.float32,
                            precision=lax.Precision.DEFAULT) + bm2_ref[:]


_final_call = pl.pallas_call(
    _final_body,
    out_shape=jax.ShapeDtypeStruct((N, D), jnp.float32),
)


def _pad_mat(w_t, rows, cols):
    return jnp.zeros((rows, cols), jnp.float32).at[: w_t.shape[0], : w_t.shape[1]].set(w_t)


def _pad_vec(b, cols):
    return jnp.zeros((1, cols), jnp.float32).at[0, : b.shape[0]].set(b)


def kernel(h, edge_index, e, emb,
           Wl0, bl0, Wr0, g0, be0,
           Wl1, bl1, Wr1, g1, be1,
           Wl2, bl2, Wr2, g2, be2,
           Wl3, bl3, Wr3, g3, be3,
           Wm0, bm0, Wm1, bm1, Wm2, bm2):
    src = edge_index[0].astype(jnp.int32)
    dst = edge_index[1].astype(jnp.int32)
    pad = E_PAD - E
    src_p = jnp.concatenate([src, jnp.zeros((pad,), jnp.int32)])
    dst_p = jnp.concatenate([dst, jnp.full((pad,), N, jnp.int32)])

    x = _embed_call(h.astype(jnp.int32)[:, None], emb)
    dacc = _deg_call(dst_p)

    layers = [(Wl0, bl0, Wr0, g0, be0), (Wl1, bl1, Wr1, g1, be1),
              (Wl2, bl2, Wr2, g2, be2)]
    for (Wl, bl, Wr, g, be) in layers:
        sacc = _agg_call(x, src_p, dst_p)
        x = _dense_call(sacc, dacc, x, Wl.T, bl[None, :], Wr.T, g[None, :],
                        be[None, :])

    sacc = _agg_call(x, src_p, dst_p)
    y = _final_call(sacc, dacc, x, Wl3.T, bl3[None, :], Wr3.T, g3[None, :],
                    be3[None, :],
                    _pad_mat(Wm0.T, D, D), _pad_vec(bm0, D),
                    _pad_mat(Wm1.T, D, D), _pad_vec(bm1, D),
                    _pad_mat(Wm2.T, D, D), _pad_vec(bm2, D))
    return y[:, : bm2.shape[0]]
